# baseline (speedup 1.0000x reference)
import functools

import jax
import jax.numpy as jnp
from jax.experimental import pallas as pl
from jax.experimental.pallas import tpu as pltpu

_K = 30
_LANES = 128
_BISECT = 18


def _emb_kernel(n_valid, x_ref, w0_ref, b0_ref, w1_ref, b1_ref, emb_ref):
    x = x_ref[...]
    h = jax.lax.dot_general(x, w0_ref[...], (((1,), (1,)), ((), ())),
                            preferred_element_type=jnp.float32)
    h = h + b0_ref[...]
    h = jnp.maximum(h, 0.0)
    h = jax.lax.dot_general(h, w1_ref[...], (((1,), (1,)), ((), ())),
                            preferred_element_type=jnp.float32)
    h = h + b1_ref[...]
    nrm = jnp.maximum(jnp.sqrt(jnp.sum(h * h, axis=1, keepdims=True)), 1e-12)
    row = jax.lax.broadcasted_iota(jnp.int32, h.shape, 0)
    emb_ref[...] = jnp.where(row < n_valid, h / nrm, 0.0)


def _sim_kernel(n_valid, embr_ref, emba_ref, out_ref):
    er = embr_ref[...]
    ea = emba_ref[...]
    sim = jax.lax.dot_general(er, ea, (((1,), (1,)), ((), ())),
                              preferred_element_type=jnp.float32)
    rb, np_ = sim.shape
    ngrp = np_ // _LANES

    lane_max = jnp.max(sim.reshape(rb, ngrp, _LANES), axis=1)

    def knock(_, m):
        mx = jnp.max(m, axis=1, keepdims=True)
        return jnp.where(m == mx, -3.0, m)

    t_hi = jnp.max(lane_max, axis=1, keepdims=True) + 1e-5
    t_lo = jnp.max(jax.lax.fori_loop(0, _K, knock, lane_max),
                   axis=1, keepdims=True)

    ones = jnp.ones((np_, 8), jnp.float32)

    def bisect(_, carry):
        lo, hi = carry
        tm = 0.5 * (lo + hi)
        mask = (sim >= tm).astype(jnp.float32)
        cnt = jax.lax.dot_general(mask, ones, (((1,), (0,)), ((), ())),
                                  preferred_element_type=jnp.float32)[:, :1]
        ge = cnt >= (_K + 1)
        return jnp.where(ge, tm, lo), jnp.where(ge, hi, tm)

    t_lo, _ = jax.lax.fori_loop(0, _BISECT, bisect, (t_lo, t_hi))

    thr = jnp.maximum(t_lo, 0.0)
    keep = sim[:, :n_valid]
    out_ref[...] = jnp.where(keep >= thr, keep, 0.0)


def kernel(features, W0, b0, W1, b1):
    n, d = features.shape
    np_ = ((n + _LANES - 1) // _LANES) * _LANES
    rb = 200 if n % 200 == 0 else n

    fpad = jnp.pad(features, ((0, np_ - n), (0, 0)))

    emb = pl.pallas_call(
        functools.partial(_emb_kernel, n),
        out_shape=jax.ShapeDtypeStruct((np_, d), jnp.float32),
    )(fpad, W0, b0.reshape(1, d), W1, b1.reshape(1, d))

    out = pl.pallas_call(
        functools.partial(_sim_kernel, n),
        grid=(n // rb,),
        in_specs=[
            pl.BlockSpec((rb, d), lambda i: (i, 0)),
            pl.BlockSpec((np_, d), lambda i: (0, 0)),
        ],
        out_specs=pl.BlockSpec((rb, n), lambda i: (i, 0)),
        out_shape=jax.ShapeDtypeStruct((n, n), jnp.float32),
        compiler_params=pltpu.CompilerParams(
            dimension_semantics=("parallel",)),
    )(emb, emb)
    return out


# speedup vs baseline: 1.2317x; 1.2317x over previous
import functools

import jax
import jax.numpy as jnp
from jax.experimental import pallas as pl
from jax.experimental.pallas import tpu as pltpu

_K = 30
_LANES = 128
_BISECT = 18


def _emb_kernel(n_valid, x_ref, w0_ref, b0_ref, w1_ref, b1_ref, emb_ref):
    x = x_ref[...]
    h = jax.lax.dot_general(x, w0_ref[...], (((1,), (1,)), ((), ())),
                            preferred_element_type=jnp.float32)
    h = h + b0_ref[...]
    h = jnp.maximum(h, 0.0)
    h = jax.lax.dot_general(h, w1_ref[...], (((1,), (1,)), ((), ())),
                            preferred_element_type=jnp.float32)
    h = h + b1_ref[...]
    nrm = jnp.maximum(jnp.sqrt(jnp.sum(h * h, axis=1, keepdims=True)), 1e-12)
    row = jax.lax.broadcasted_iota(jnp.int32, h.shape, 0)
    emb_ref[...] = jnp.where(row < n_valid, h / nrm, 0.0)


def _sim_kernel(n_valid, embr_ref, emba_ref, out_ref):
    er = embr_ref[...]
    ea = emba_ref[...]
    sim = jax.lax.dot_general(er, ea, (((1,), (1,)), ((), ())),
                              preferred_element_type=jnp.float32)
    rb, np_ = sim.shape
    ngrp = np_ // _LANES

    lane_max = jnp.max(sim.reshape(rb, ngrp, _LANES), axis=1)

    def knock(_, m):
        mx = jnp.max(m, axis=1, keepdims=True)
        return jnp.where(m == mx, -3.0, m)

    t_hi = jnp.max(lane_max, axis=1, keepdims=True) + 1e-5
    t_lo = jnp.max(jax.lax.fori_loop(0, _K, knock, lane_max),
                   axis=1, keepdims=True)

    def bisect(_, carry):
        lo, hi = carry
        tm = 0.5 * (lo + hi)
        cnt = jnp.sum(jnp.where(sim >= tm, 1.0, 0.0), axis=1, keepdims=True)
        ge = cnt >= (_K + 1)
        return jnp.where(ge, tm, lo), jnp.where(ge, hi, tm)

    t_lo, _ = jax.lax.fori_loop(0, _BISECT, bisect, (t_lo, t_hi))

    thr = jnp.maximum(t_lo, 0.0)
    keep = sim[:, :n_valid]
    out_ref[...] = jnp.where(keep >= thr, keep, 0.0)


def kernel(features, W0, b0, W1, b1):
    n, d = features.shape
    np_ = ((n + _LANES - 1) // _LANES) * _LANES
    rb = 200 if n % 200 == 0 else n

    fpad = jnp.pad(features, ((0, np_ - n), (0, 0)))

    emb = pl.pallas_call(
        functools.partial(_emb_kernel, n),
        out_shape=jax.ShapeDtypeStruct((np_, d), jnp.float32),
    )(fpad, W0, b0.reshape(1, d), W1, b1.reshape(1, d))

    out = pl.pallas_call(
        functools.partial(_sim_kernel, n),
        grid=(n // rb,),
        in_specs=[
            pl.BlockSpec((rb, d), lambda i: (i, 0)),
            pl.BlockSpec((np_, d), lambda i: (0, 0)),
        ],
        out_specs=pl.BlockSpec((rb, n), lambda i: (i, 0)),
        out_shape=jax.ShapeDtypeStruct((n, n), jnp.float32),
        compiler_params=pltpu.CompilerParams(
            dimension_semantics=("parallel",)),
    )(emb, emb)
    return out


# speedup vs baseline: 1.3938x; 1.1316x over previous
import functools

import jax
import jax.numpy as jnp
from jax.experimental import pallas as pl
from jax.experimental.pallas import tpu as pltpu

_K = 30
_LANES = 128
_BISECT = 10


def _emb_kernel(n_valid, x_ref, w0_ref, b0_ref, w1_ref, b1_ref, emb_ref):
    x = x_ref[...]
    h = jax.lax.dot_general(x, w0_ref[...], (((1,), (1,)), ((), ())),
                            preferred_element_type=jnp.float32)
    h = h + b0_ref[...]
    h = jnp.maximum(h, 0.0)
    h = jax.lax.dot_general(h, w1_ref[...], (((1,), (1,)), ((), ())),
                            preferred_element_type=jnp.float32)
    h = h + b1_ref[...]
    nrm = jnp.maximum(jnp.sqrt(jnp.sum(h * h, axis=1, keepdims=True)), 1e-12)
    row = jax.lax.broadcasted_iota(jnp.int32, h.shape, 0)
    emb_ref[...] = jnp.where(row < n_valid, h / nrm, 0.0)


def _sim_kernel(n_valid, embr_ref, emba_ref, out_ref):
    er = embr_ref[...]
    ea = emba_ref[...]
    sim = jax.lax.dot_general(er, ea, (((1,), (1,)), ((), ())),
                              preferred_element_type=jnp.float32)
    rb, np_ = sim.shape
    ngrp = np_ // _LANES

    lane_max = jnp.max(sim.reshape(rb, ngrp, _LANES), axis=1)

    def knock(_, m):
        mx = jnp.max(m, axis=1, keepdims=True)
        return jnp.where(m == mx, -3.0, m)

    t_hi = jnp.max(lane_max, axis=1, keepdims=True) + 1e-5
    t_lo = jnp.max(jax.lax.fori_loop(0, _K, knock, lane_max),
                   axis=1, keepdims=True)

    def bisect(_, carry):
        lo, hi = carry
        tm = 0.5 * (lo + hi)
        cnt = jnp.sum(jnp.where(sim >= tm, 1.0, 0.0), axis=1, keepdims=True)
        ge = cnt >= (_K + 1)
        return jnp.where(ge, tm, lo), jnp.where(ge, hi, tm)

    t_lo, _ = jax.lax.fori_loop(0, _BISECT, bisect, (t_lo, t_hi))

    cnt = jnp.sum(jnp.where(sim >= t_lo, 1.0, 0.0), axis=1, keepdims=True)
    e = cnt - (_K + 1)

    m1 = jnp.min(jnp.where(sim >= t_lo, sim, 3.0), axis=1, keepdims=True)
    m2 = jnp.min(jnp.where(sim > m1, sim, 3.0), axis=1, keepdims=True)
    m3 = jnp.min(jnp.where(sim > m2, sim, 3.0), axis=1, keepdims=True)
    m4 = jnp.min(jnp.where(sim > m3, sim, 3.0), axis=1, keepdims=True)
    thr = jnp.where(e <= 0.0, m1,
                    jnp.where(e == 1.0, m2,
                              jnp.where(e == 2.0, m3,
                                        jnp.where(e == 3.0, m4, t_lo))))

    thr = jnp.maximum(thr, 0.0)
    keep = sim[:, :n_valid]
    out_ref[...] = jnp.where(keep >= thr, keep, 0.0)


def kernel(features, W0, b0, W1, b1):
    n, d = features.shape
    np_ = ((n + _LANES - 1) // _LANES) * _LANES
    rb = 200 if n % 200 == 0 else n

    fpad = jnp.pad(features, ((0, np_ - n), (0, 0)))

    emb = pl.pallas_call(
        functools.partial(_emb_kernel, n),
        out_shape=jax.ShapeDtypeStruct((np_, d), jnp.float32),
    )(fpad, W0, b0.reshape(1, d), W1, b1.reshape(1, d))

    out = pl.pallas_call(
        functools.partial(_sim_kernel, n),
        grid=(n // rb,),
        in_specs=[
            pl.BlockSpec((rb, d), lambda i: (i, 0)),
            pl.BlockSpec((np_, d), lambda i: (0, 0)),
        ],
        out_specs=pl.BlockSpec((rb, n), lambda i: (i, 0)),
        out_shape=jax.ShapeDtypeStruct((n, n), jnp.float32),
        compiler_params=pltpu.CompilerParams(
            dimension_semantics=("parallel",)),
    )(emb, emb)
    return out


# speedup vs baseline: 2.6141x; 1.8756x over previous
import functools

import jax
import jax.numpy as jnp
from jax.experimental import pallas as pl
from jax.experimental.pallas import tpu as pltpu

_K = 30
_LANES = 128
_BISECT = 24


def _emb_kernel(n_valid, x_ref, w0_ref, b0_ref, w1_ref, b1_ref, emb_ref):
    x = x_ref[...]
    h = jax.lax.dot_general(x, w0_ref[...], (((1,), (1,)), ((), ())),
                            preferred_element_type=jnp.float32)
    h = h + b0_ref[...]
    h = jnp.maximum(h, 0.0)
    h = jax.lax.dot_general(h, w1_ref[...], (((1,), (1,)), ((), ())),
                            preferred_element_type=jnp.float32)
    h = h + b1_ref[...]
    nrm = jnp.maximum(jnp.sqrt(jnp.sum(h * h, axis=1, keepdims=True)), 1e-12)
    row = jax.lax.broadcasted_iota(jnp.int32, h.shape, 0)
    emb_ref[...] = jnp.where(row < n_valid, h / nrm, 0.0)


def _sim_kernel(n_valid, embr_ref, emba_ref, out_ref):
    er = embr_ref[...]
    ea = emba_ref[...]
    sim = jax.lax.dot_general(er, ea, (((1,), (1,)), ((), ())),
                              preferred_element_type=jnp.float32)
    rb, np_ = sim.shape
    ngrp = np_ // _LANES

    v = sim.reshape(rb, ngrp, _LANES)
    a = jnp.max(v, axis=1)
    v2 = jnp.where(v == a[:, None, :], -3.0, v)
    b = jnp.max(v2, axis=1)
    c = jnp.max(jnp.where(v2 == b[:, None, :], -3.0, v2), axis=1)
    cand = jnp.concatenate([a, b, c], axis=1)

    def bisect(_, carry):
        lo, hi = carry
        tm = 0.5 * (lo + hi)
        cnt_c = jnp.sum(jnp.where(cand >= tm, 1.0, 0.0), axis=1, keepdims=True)
        ge = cnt_c >= (_K + 1)
        return jnp.where(ge, tm, lo), jnp.where(ge, hi, tm)

    t_lo0 = jnp.full((rb, 1), -1.001, jnp.float32)
    t_hi0 = jnp.full((rb, 1), 1.001, jnp.float32)
    t_lo, _ = jax.lax.fori_loop(0, _BISECT, bisect, (t_lo0, t_hi0))

    cnt = jnp.sum(jnp.where(sim >= t_lo, 1.0, 0.0), axis=1, keepdims=True)
    e = cnt - (_K + 1)

    m1 = jnp.min(jnp.where(sim >= t_lo, sim, 3.0), axis=1, keepdims=True)
    m2 = jnp.min(jnp.where(sim > m1, sim, 3.0), axis=1, keepdims=True)
    m3 = jnp.min(jnp.where(sim > m2, sim, 3.0), axis=1, keepdims=True)
    m4 = jnp.min(jnp.where(sim > m3, sim, 3.0), axis=1, keepdims=True)
    thr = jnp.where(e <= 0.0, m1,
                    jnp.where(e == 1.0, m2,
                              jnp.where(e == 2.0, m3,
                                        jnp.where(e == 3.0, m4, t_lo))))

    thr = jnp.maximum(thr, 0.0)
    keep = sim[:, :n_valid]
    out_ref[...] = jnp.where(keep >= thr, keep, 0.0)


def kernel(features, W0, b0, W1, b1):
    n, d = features.shape
    np_ = ((n + _LANES - 1) // _LANES) * _LANES
    rb = 200 if n % 200 == 0 else n

    fpad = jnp.pad(features, ((0, np_ - n), (0, 0)))

    emb = pl.pallas_call(
        functools.partial(_emb_kernel, n),
        out_shape=jax.ShapeDtypeStruct((np_, d), jnp.float32),
    )(fpad, W0, b0.reshape(1, d), W1, b1.reshape(1, d))

    out = pl.pallas_call(
        functools.partial(_sim_kernel, n),
        grid=(n // rb,),
        in_specs=[
            pl.BlockSpec((rb, d), lambda i: (i, 0)),
            pl.BlockSpec((np_, d), lambda i: (0, 0)),
        ],
        out_specs=pl.BlockSpec((rb, n), lambda i: (i, 0)),
        out_shape=jax.ShapeDtypeStruct((n, n), jnp.float32),
        compiler_params=pltpu.CompilerParams(
            dimension_semantics=("parallel",)),
    )(emb, emb)
    return out


# speedup vs baseline: 2.8282x; 1.0819x over previous
import functools

import jax
import jax.numpy as jnp
from jax.experimental import pallas as pl
from jax.experimental.pallas import tpu as pltpu

_K = 30
_LANES = 128
_BISECT = 16


def _emb_kernel(n_valid, x_ref, w0_ref, b0_ref, w1_ref, b1_ref, emb_ref):
    x = x_ref[...]
    h = jax.lax.dot_general(x, w0_ref[...], (((1,), (1,)), ((), ())),
                            preferred_element_type=jnp.float32)
    h = h + b0_ref[...]
    h = jnp.maximum(h, 0.0)
    h = jax.lax.dot_general(h, w1_ref[...], (((1,), (1,)), ((), ())),
                            preferred_element_type=jnp.float32)
    h = h + b1_ref[...]
    nrm = jnp.maximum(jnp.sqrt(jnp.sum(h * h, axis=1, keepdims=True)), 1e-12)
    row = jax.lax.broadcasted_iota(jnp.int32, h.shape, 0)
    emb_ref[...] = jnp.where(row < n_valid, h / nrm, 0.0)


def _sim_kernel(n_valid, embr_ref, emba_ref, out_ref):
    er = embr_ref[...]
    ea = emba_ref[...]
    sim = jax.lax.dot_general(er, ea, (((1,), (1,)), ((), ())),
                              preferred_element_type=jnp.float32)
    rb, np_ = sim.shape
    ngrp = np_ // _LANES

    v = sim.reshape(rb, ngrp, _LANES)
    a = jnp.max(v, axis=1)
    v2 = jnp.where(v == a[:, None, :], -3.0, v)
    b = jnp.max(v2, axis=1)
    c = jnp.max(jnp.where(v2 == b[:, None, :], -3.0, v2), axis=1)
    cand = jnp.concatenate([a, b, c], axis=1)

    def bisect(_, carry):
        lo, hi = carry
        tm = 0.5 * (lo + hi)
        cnt_c = jnp.sum(jnp.where(cand >= tm, 1.0, 0.0), axis=1, keepdims=True)
        ge = cnt_c >= (_K + 1)
        return jnp.where(ge, tm, lo), jnp.where(ge, hi, tm)

    t_lo0 = jnp.full((rb, 1), -1.001, jnp.float32)
    t_hi0 = jnp.full((rb, 1), 1.001, jnp.float32)
    t_lo, _ = jax.lax.fori_loop(0, _BISECT, bisect, (t_lo0, t_hi0))

    cnt = jnp.sum(jnp.where(sim >= t_lo, 1.0, 0.0), axis=1, keepdims=True)
    e = cnt - (_K + 1)

    m1 = jnp.min(jnp.where(sim >= t_lo, sim, 3.0), axis=1, keepdims=True)
    m2 = jnp.min(jnp.where(sim > m1, sim, 3.0), axis=1, keepdims=True)
    m3 = jnp.min(jnp.where(sim > m2, sim, 3.0), axis=1, keepdims=True)
    m4 = jnp.min(jnp.where(sim > m3, sim, 3.0), axis=1, keepdims=True)
    thr = jnp.where(e <= 0.0, m1,
                    jnp.where(e == 1.0, m2,
                              jnp.where(e == 2.0, m3,
                                        jnp.where(e == 3.0, m4, t_lo))))

    thr = jnp.maximum(thr, 0.0)
    keep = sim[:, :n_valid]
    out_ref[...] = jnp.where(keep >= thr, keep, 0.0)


def kernel(features, W0, b0, W1, b1):
    n, d = features.shape
    np_ = ((n + _LANES - 1) // _LANES) * _LANES
    rb = 200 if n % 200 == 0 else n

    fpad = jnp.pad(features, ((0, np_ - n), (0, 0)))

    emb = pl.pallas_call(
        functools.partial(_emb_kernel, n),
        out_shape=jax.ShapeDtypeStruct((np_, d), jnp.float32),
    )(fpad, W0, b0.reshape(1, d), W1, b1.reshape(1, d))

    out = pl.pallas_call(
        functools.partial(_sim_kernel, n),
        grid=(n // rb,),
        in_specs=[
            pl.BlockSpec((rb, d), lambda i: (i, 0)),
            pl.BlockSpec((np_, d), lambda i: (0, 0)),
        ],
        out_specs=pl.BlockSpec((rb, n), lambda i: (i, 0)),
        out_shape=jax.ShapeDtypeStruct((n, n), jnp.float32),
        compiler_params=pltpu.CompilerParams(
            dimension_semantics=("parallel",)),
    )(emb, emb)
    return out


# speedup vs baseline: 4.4160x; 1.5614x over previous
import functools

import jax
import jax.numpy as jnp
from jax.experimental import pallas as pl
from jax.experimental.pallas import tpu as pltpu

_K = 30
_LANES = 128
_BISECT = 16


def _emb_kernel(n_valid, x_ref, w0_ref, b0_ref, w1_ref, b1_ref, emb_ref):
    x = x_ref[...]
    h = jax.lax.dot_general(x, w0_ref[...], (((1,), (1,)), ((), ())),
                            preferred_element_type=jnp.float32)
    h = h + b0_ref[...]
    h = jnp.maximum(h, 0.0)
    h = jax.lax.dot_general(h, w1_ref[...], (((1,), (1,)), ((), ())),
                            preferred_element_type=jnp.float32)
    h = h + b1_ref[...]
    nrm = jnp.maximum(jnp.sqrt(jnp.sum(h * h, axis=1, keepdims=True)), 1e-12)
    row = jax.lax.broadcasted_iota(jnp.int32, h.shape, 0)
    emb_ref[...] = jnp.where(row < n_valid, h / nrm, 0.0)


def _sim_kernel(n_valid, embr_ref, emba_ref, out_ref):
    er = embr_ref[...]
    ea = emba_ref[...]
    sim = jax.lax.dot_general(er, ea, (((1,), (1,)), ((), ())),
                              preferred_element_type=jnp.float32)
    rb, np_ = sim.shape
    ngrp = np_ // _LANES

    neg = jnp.full((rb, _LANES), -3.0, jnp.float32)
    a, b, c = neg, neg, neg
    for g in range(ngrp):
        x = sim[:, g * _LANES:(g + 1) * _LANES]
        t = jnp.minimum(a, x)
        a = jnp.maximum(a, x)
        u = jnp.minimum(b, t)
        b = jnp.maximum(b, t)
        c = jnp.maximum(c, u)
    cand = jnp.concatenate([a, b, c], axis=1)

    def bisect(_, carry):
        lo, hi = carry
        tm = 0.5 * (lo + hi)
        cnt_c = jnp.sum(jnp.where(cand >= tm, 1.0, 0.0), axis=1, keepdims=True)
        ge = cnt_c >= (_K + 1)
        return jnp.where(ge, tm, lo), jnp.where(ge, hi, tm)

    t_lo0 = jnp.full((rb, 1), -1.001, jnp.float32)
    t_hi0 = jnp.full((rb, 1), 1.001, jnp.float32)
    t_lo, _ = jax.lax.fori_loop(0, _BISECT, bisect, (t_lo0, t_hi0))

    pos = jnp.full((rb, _LANES), 3.0, jnp.float32)
    p, q, r = pos, pos, pos
    acc = jnp.zeros((rb, _LANES), jnp.float32)
    for g in range(ngrp):
        xg = sim[:, g * _LANES:(g + 1) * _LANES]
        m = xg >= t_lo
        x = jnp.where(m, xg, 3.0)
        acc = acc + jnp.where(m, 1.0, 0.0)
        t = jnp.maximum(p, x)
        p = jnp.minimum(p, x)
        u = jnp.maximum(q, t)
        q = jnp.minimum(q, t)
        r = jnp.minimum(r, u)
    e = jnp.sum(acc, axis=1, keepdims=True) - (_K + 1)

    candb = jnp.concatenate([p, q, r], axis=1)
    mn1 = jnp.min(candb, axis=1, keepdims=True)
    candb = jnp.where(candb == mn1, 3.0, candb)
    mn2 = jnp.min(candb, axis=1, keepdims=True)
    candb = jnp.where(candb == mn2, 3.0, candb)
    mn3 = jnp.min(candb, axis=1, keepdims=True)
    candb = jnp.where(candb == mn3, 3.0, candb)
    mn4 = jnp.min(candb, axis=1, keepdims=True)
    thr = jnp.where(e <= 0.0, mn1,
                    jnp.where(e == 1.0, mn2,
                              jnp.where(e == 2.0, mn3,
                                        jnp.where(e == 3.0, mn4, t_lo))))

    thr = jnp.maximum(thr, 0.0)
    keep = sim[:, :n_valid]
    out_ref[...] = jnp.where(keep >= thr, keep, 0.0)


def kernel(features, W0, b0, W1, b1):
    n, d = features.shape
    np_ = ((n + _LANES - 1) // _LANES) * _LANES
    rb = 200 if n % 200 == 0 else n

    fpad = jnp.pad(features, ((0, np_ - n), (0, 0)))

    emb = pl.pallas_call(
        functools.partial(_emb_kernel, n),
        out_shape=jax.ShapeDtypeStruct((np_, d), jnp.float32),
    )(fpad, W0, b0.reshape(1, d), W1, b1.reshape(1, d))

    out = pl.pallas_call(
        functools.partial(_sim_kernel, n),
        grid=(n // rb,),
        in_specs=[
            pl.BlockSpec((rb, d), lambda i: (i, 0)),
            pl.BlockSpec((np_, d), lambda i: (0, 0)),
        ],
        out_specs=pl.BlockSpec((rb, n), lambda i: (i, 0)),
        out_shape=jax.ShapeDtypeStruct((n, n), jnp.float32),
        compiler_params=pltpu.CompilerParams(
            dimension_semantics=("parallel",)),
    )(emb, emb)
    return out


# speedup vs baseline: 4.6980x; 1.0639x over previous
import functools

import jax
import jax.numpy as jnp
from jax.experimental import pallas as pl
from jax.experimental.pallas import tpu as pltpu

_K = 30
_LANES = 128
_BISECT = 16


def _emb_kernel(n_valid, x_ref, w0_ref, b0_ref, w1_ref, b1_ref, emb_ref):
    x = x_ref[...]
    h = jax.lax.dot_general(x, w0_ref[...], (((1,), (1,)), ((), ())),
                            preferred_element_type=jnp.float32)
    h = h + b0_ref[...]
    h = jnp.maximum(h, 0.0)
    h = jax.lax.dot_general(h, w1_ref[...], (((1,), (1,)), ((), ())),
                            preferred_element_type=jnp.float32)
    h = h + b1_ref[...]
    nrm = jnp.maximum(jnp.sqrt(jnp.sum(h * h, axis=1, keepdims=True)), 1e-12)
    row = jax.lax.broadcasted_iota(jnp.int32, h.shape, 0)
    emb_ref[...] = jnp.where(row < n_valid, h / nrm, 0.0)


def _sim_kernel(n_valid, embr_ref, emba_ref, out_ref):
    er = embr_ref[...]
    ea = emba_ref[...]
    sim = jax.lax.dot_general(er, ea, (((1,), (1,)), ((), ())),
                              preferred_element_type=jnp.float32)
    rb, np_ = sim.shape
    ngrp = np_ // _LANES

    neg = jnp.full((rb, _LANES), -3.0, jnp.float32)
    a, b, c = neg, neg, neg
    for g in range(ngrp):
        x = sim[:, g * _LANES:(g + 1) * _LANES]
        t = jnp.minimum(a, x)
        a = jnp.maximum(a, x)
        u = jnp.minimum(b, t)
        b = jnp.maximum(b, t)
        c = jnp.maximum(c, u)
    cand = jnp.concatenate([a, b, c], axis=1)

    def bisect(_, carry):
        lo, hi = carry
        tm = 0.5 * (lo + hi)
        cnt_c = jnp.sum(jnp.where(cand >= tm, 1.0, 0.0), axis=1, keepdims=True)
        ge = cnt_c >= (_K + 1)
        return jnp.where(ge, tm, lo), jnp.where(ge, hi, tm)

    t_lo0 = jnp.min(c, axis=1, keepdims=True)
    t_hi0 = jnp.max(a, axis=1, keepdims=True) + 1e-5
    t_lo, _ = jax.lax.fori_loop(0, _BISECT, bisect, (t_lo0, t_hi0))

    pos = jnp.full((rb, _LANES), 3.0, jnp.float32)
    p, q, r = pos, pos, pos
    acc = jnp.zeros((rb, _LANES), jnp.float32)
    for g in range(ngrp):
        xg = sim[:, g * _LANES:(g + 1) * _LANES]
        m = xg >= t_lo
        x = jnp.where(m, xg, 3.0)
        acc = acc + jnp.where(m, 1.0, 0.0)
        t = jnp.maximum(p, x)
        p = jnp.minimum(p, x)
        u = jnp.maximum(q, t)
        q = jnp.minimum(q, t)
        r = jnp.minimum(r, u)
    e = jnp.sum(acc, axis=1, keepdims=True) - (_K + 1)

    candb = jnp.concatenate([p, q, r], axis=1)
    mn1 = jnp.min(candb, axis=1, keepdims=True)
    candb = jnp.where(candb == mn1, 3.0, candb)
    mn2 = jnp.min(candb, axis=1, keepdims=True)
    candb = jnp.where(candb == mn2, 3.0, candb)
    mn3 = jnp.min(candb, axis=1, keepdims=True)
    candb = jnp.where(candb == mn3, 3.0, candb)
    mn4 = jnp.min(candb, axis=1, keepdims=True)
    thr = jnp.where(e <= 0.0, mn1,
                    jnp.where(e == 1.0, mn2,
                              jnp.where(e == 2.0, mn3,
                                        jnp.where(e == 3.0, mn4, t_lo))))

    thr = jnp.maximum(thr, 0.0)
    keep = sim[:, :n_valid]
    out_ref[...] = jnp.where(keep >= thr, keep, 0.0)


def kernel(features, W0, b0, W1, b1):
    n, d = features.shape
    np_ = ((n + _LANES - 1) // _LANES) * _LANES
    rb = 400 if n % 400 == 0 else n

    fpad = jnp.pad(features, ((0, np_ - n), (0, 0)))

    emb = pl.pallas_call(
        functools.partial(_emb_kernel, n),
        out_shape=jax.ShapeDtypeStruct((np_, d), jnp.float32),
    )(fpad, W0, b0.reshape(1, d), W1, b1.reshape(1, d))

    out = pl.pallas_call(
        functools.partial(_sim_kernel, n),
        grid=(n // rb,),
        in_specs=[
            pl.BlockSpec((rb, d), lambda i: (i, 0)),
            pl.BlockSpec((np_, d), lambda i: (0, 0)),
        ],
        out_specs=pl.BlockSpec((rb, n), lambda i: (i, 0)),
        out_shape=jax.ShapeDtypeStruct((n, n), jnp.float32),
        compiler_params=pltpu.CompilerParams(
            dimension_semantics=("parallel",)),
    )(emb, emb)
    return out


# speedup vs baseline: 5.1030x; 1.0862x over previous
import functools

import jax
import jax.numpy as jnp
from jax.experimental import pallas as pl
from jax.experimental.pallas import tpu as pltpu

_K = 30
_LANES = 128
_BISECT = 16


def _emb_kernel(n_valid, x_ref, w0_ref, b0_ref, w1_ref, b1_ref, emb_ref):
    x = x_ref[...]
    h = jax.lax.dot_general(x, w0_ref[...], (((1,), (1,)), ((), ())),
                            preferred_element_type=jnp.float32)
    h = h + b0_ref[...]
    h = jnp.maximum(h, 0.0)
    h = jax.lax.dot_general(h, w1_ref[...], (((1,), (1,)), ((), ())),
                            preferred_element_type=jnp.float32)
    h = h + b1_ref[...]
    nrm = jnp.maximum(jnp.sqrt(jnp.sum(h * h, axis=1, keepdims=True)), 1e-12)
    row = jax.lax.broadcasted_iota(jnp.int32, h.shape, 0)
    emb_ref[...] = jnp.where(row < n_valid, h / nrm, 0.0)


def _sim_kernel(n_valid, embr_ref, emba_ref, out_ref):
    er = embr_ref[...]
    ea = emba_ref[...]
    sim = jax.lax.dot_general(er, ea, (((1,), (1,)), ((), ())),
                              preferred_element_type=jnp.float32)
    rb, np_ = sim.shape
    ngrp = np_ // _LANES

    neg = jnp.full((rb, _LANES), -3.0, jnp.float32)
    a, b, c = neg, neg, neg
    for g in range(ngrp):
        x = sim[:, g * _LANES:(g + 1) * _LANES]
        t = jnp.minimum(a, x)
        a = jnp.maximum(a, x)
        u = jnp.minimum(b, t)
        b = jnp.maximum(b, t)
        c = jnp.maximum(c, u)
    cand = jnp.concatenate([a, b, c], axis=1)

    def bisect(_, carry):
        lo, hi = carry
        tm = 0.5 * (lo + hi)
        cnt_c = jnp.sum(jnp.where(cand >= tm, 1.0, 0.0), axis=1, keepdims=True)
        ge = cnt_c >= (_K + 1)
        return jnp.where(ge, tm, lo), jnp.where(ge, hi, tm)

    t_lo0 = jnp.min(c, axis=1, keepdims=True)
    t_hi0 = jnp.max(a, axis=1, keepdims=True) + 1e-5
    t_lo, _ = jax.lax.fori_loop(0, _BISECT, bisect, (t_lo0, t_hi0))

    pos = jnp.full((rb, _LANES), 3.0, jnp.float32)
    p, q = pos, pos
    acc = jnp.zeros((rb, _LANES), jnp.float32)
    for g in range(ngrp):
        xg = sim[:, g * _LANES:(g + 1) * _LANES]
        m = xg >= t_lo
        x = jnp.where(m, xg, 3.0)
        acc = acc + jnp.where(m, 1.0, 0.0)
        t = jnp.maximum(p, x)
        p = jnp.minimum(p, x)
        q = jnp.minimum(q, t)
    e = jnp.sum(acc, axis=1, keepdims=True) - (_K + 1)

    candb = jnp.concatenate([p, q], axis=1)
    mn1 = jnp.min(candb, axis=1, keepdims=True)
    candb = jnp.where(candb == mn1, 3.0, candb)
    mn2 = jnp.min(candb, axis=1, keepdims=True)
    candb = jnp.where(candb == mn2, 3.0, candb)
    mn3 = jnp.min(candb, axis=1, keepdims=True)
    thr = jnp.where(e <= 0.0, mn1,
                    jnp.where(e == 1.0, mn2,
                              jnp.where(e == 2.0, mn3, t_lo)))

    thr = jnp.maximum(thr, 0.0)
    keep = sim[:, :n_valid]
    out_ref[...] = jnp.where(keep >= thr, keep, 0.0)


def kernel(features, W0, b0, W1, b1):
    n, d = features.shape
    np_ = ((n + _LANES - 1) // _LANES) * _LANES
    rb = 400 if n % 400 == 0 else n

    fpad = jnp.pad(features, ((0, np_ - n), (0, 0)))

    emb = pl.pallas_call(
        functools.partial(_emb_kernel, n),
        out_shape=jax.ShapeDtypeStruct((np_, d), jnp.float32),
    )(fpad, W0, b0.reshape(1, d), W1, b1.reshape(1, d))

    out = pl.pallas_call(
        functools.partial(_sim_kernel, n),
        grid=(n // rb,),
        in_specs=[
            pl.BlockSpec((rb, d), lambda i: (i, 0)),
            pl.BlockSpec((np_, d), lambda i: (0, 0)),
        ],
        out_specs=pl.BlockSpec((rb, n), lambda i: (i, 0)),
        out_shape=jax.ShapeDtypeStruct((n, n), jnp.float32),
        compiler_params=pltpu.CompilerParams(
            dimension_semantics=("parallel",)),
    )(emb, emb)
    return out
